# baseline (speedup 1.0000x reference)
import functools

import jax
import jax.numpy as jnp
from jax import lax
from jax.experimental import pallas as pl
from jax.experimental.pallas import tpu as pltpu
from jax.experimental.pallas import tpu_sc as plsc

N = 10000
E = 320000
ET = E + N
NC = 2
NS = 16
LANES = 16
BLK = 128
NCH = 3
NQ = 14
KBC = 4 * NQ
KB = NCH * KBC
EPT = KB * BLK
EPAD = NS * EPT
NPT = 640
C = 64
CL = C // LANES


def _sc_layer_fn(S, T, K):
    mesh = plsc.VectorSubcoreMesh(
        core_axis_name="c", subcore_axis_name="s",
        num_cores=NC, num_subcores=NS)

    @functools.partial(
        pl.kernel,
        out_type=jax.ShapeDtypeStruct((NC, N, C), jnp.float32),
        mesh=mesh,
        compiler_params=pltpu.CompilerParams(
            needs_layout_passes=False, use_tc_tiling_on_sc=False),
        scratch_types=[
            pltpu.VMEM((KBC, BLK), jnp.int32),
            pltpu.VMEM((KBC, BLK), jnp.int32),
            pltpu.VMEM((KBC, BLK), jnp.float32),
            pltpu.VMEM((N,), jnp.float32),
            pltpu.VMEM((N,), jnp.float32),
            pltpu.VMEM((BLK, C), jnp.float32),
            pltpu.VMEM((BLK, C), jnp.float32),
            pltpu.VMEM((BLK, C), jnp.float32),
            pltpu.VMEM((BLK, C), jnp.float32),
            pltpu.VMEM((LANES,), jnp.float32),
            pltpu.VMEM_SHARED((N,), jnp.float32),
            pltpu.VMEM_SHARED((N, C), jnp.float32),
            pltpu.SemaphoreType.DMA,
            pltpu.SemaphoreType.DMA,
            pltpu.SemaphoreType.DMA,
            pltpu.SemaphoreType.DMA,
            pltpu.SemaphoreType.DMA,
            pltpu.SemaphoreType.DMA,
            pltpu.SemaphoreType.DMA,
            pltpu.SemaphoreType.DMA,
            pltpu.SemaphoreType.DMA,
        ],
    )
    def sc_layer(h_hbm, ab_hbm, m_hbm, src_hbm, dst_hbm, bias_hbm, out_hbm,
                 srcg, dstv, w, va, vb, r0, r1, r2, r3, vmb, denom, acc,
                 g0, g1, g2, g3, s0, s1, s2, s3, dsem):
        i32 = jnp.int32
        f32 = jnp.float32
        c = lax.axis_index("c")
        t = lax.axis_index("s")
        giota = lax.iota(i32, LANES)
        base = t * NPT
        zero16 = jnp.zeros((LANES,), f32)
        bufs = [r0, r1, r2, r3]
        gsems = [g0, g1, g2, g3]
        ssems = [s0, s1, s2, s3]

        for q in range(BLK // LANES):
            w[0, pl.ds(LANES * q, LANES)] = zero16

        def zrow(r, carry):
            for q in range(CL):
                r0[r, pl.ds(LANES * q, LANES)] = zero16
            return carry

        lax.fori_loop(0, BLK, zrow, 0)
        for k in range(5):
            off = base + 128 * k

            @pl.when(off + 128 <= N)
            def _():
                pltpu.sync_copy(w.at[0], denom.at[pl.ds(off, 128)])
                pltpu.sync_copy(r0, acc.at[pl.ds(off, 128)])

        @pl.when(t == NS - 1)
        def _():
            pltpu.sync_copy(w.at[0, pl.ds(0, 16)],
                            denom.at[pl.ds(N - 16, 16)])
            pltpu.sync_copy(r0.at[pl.ds(0, 16)],
                            acc.at[pl.ds(N - 16, 16)])

        pltpu.sync_copy(ab_hbm.at[0], va)
        pltpu.sync_copy(ab_hbm.at[1], vb)
        pltpu.sync_copy(m_hbm, vmb)
        mb = vmb[...]

        plsc.subcore_barrier()

        def g_issue(j, buf, sem):
            pltpu.async_copy(h_hbm.at[srcg.at[j]], buf, sem)

        def g_drain(buf, sem):
            pltpu.make_async_copy(h_hbm.at[srcg.at[0]], buf, sem).wait()

        def s_issue(j, buf, sem):
            pltpu.async_copy(buf, acc.at[dstv.at[j]], sem, add=True)

        def s_drain(buf, sem):
            pltpu.make_async_copy(buf, acc.at[pl.ds(0, BLK)], sem).wait()

        def d_issue(j):
            pltpu.async_copy(w.at[j], denom.at[dstv.at[j]], dsem, add=True)

        def d_drain():
            pltpu.make_async_copy(w.at[0], denom.at[pl.ds(0, BLK)],
                                  dsem).wait()

        def scale(buf, j):
            jv = jnp.full((LANES,), j, i32)

            def estep(i, ev):
                e0 = i * 4
                for u in range(4):
                    av = plsc.load_gather(w, [jv, ev + u])
                    for q in range(CL):
                        sl = pl.ds(LANES * q, LANES)
                        buf[e0 + u, sl] = buf[e0 + u, sl] * av
                return ev + 4

            lax.fori_loop(0, BLK // 4, estep, jnp.zeros((LANES,), i32))

        for ch in range(NCH):
            cb = ch * KBC
            pltpu.sync_copy(src_hbm.at[t, pl.ds(cb, KBC)], srcg)
            pltpu.sync_copy(dst_hbm.at[t, pl.ds(cb, KBC)], dstv)
            ebase = t * EPT + cb * BLK

            def escomp(j, carry):
                jb = ebase + j * BLK
                for k in range(BLK // LANES):
                    sl = pl.ds(LANES * k, LANES)
                    sv = srcg[j, sl]
                    dv = dstv[j, sl]
                    e = (plsc.load_gather(va, [sv])
                         + plsc.load_gather(vb, [dv]))
                    e = jnp.where(e >= 0, e, 0.2 * e)
                    ex = jnp.exp(e - mb)
                    gid = jb + LANES * k + giota
                    ex = jnp.where(gid < ET, ex, 0.0)
                    w[j, sl] = ex
                    srcg[j, sl] = sv * S + c * T + K
                return carry

            lax.fori_loop(0, KBC, escomp, 0)

            g_issue(0, r0, g0)
            g_issue(1, r1, g1)
            g_issue(2, r2, g2)

            def quad(qq, carry):
                for i in range(4):
                    j = 4 * qq + i
                    buf, gs, ss = bufs[i], gsems[i], ssems[i]
                    pbuf = bufs[(i + 3) % 4]
                    pss = ssems[(i + 3) % 4]
                    g_drain(buf, gs)
                    scale(buf, j)
                    s_issue(j, buf, ss)
                    d_issue(j)

                    @pl.when(j > 0)
                    def _():
                        s_drain(pbuf, pss)
                        d_drain()

                    @pl.when(j + 3 < KBC)
                    def _():
                        g_issue(j + 3, pbuf, gsems[(i + 3) % 4])
                return carry

            lax.fori_loop(0, NQ, quad, 0)
            s_drain(r3, s3)
            d_drain()

        plsc.subcore_barrier()

        pltpu.sync_copy(denom, va)
        nv = jnp.where(t == NS - 1, 25, 40)

        def rstep(r, carry):
            sl = pl.ds(base + LANES * r, LANES)
            va[sl] = 1.0 / (va[sl] + 1e-16)
            return carry

        lax.fori_loop(0, nv, rstep, 0)
        pltpu.sync_copy(bias_hbm.at[c], r1.at[0])
        bvecs = [r1[0, pl.ds(LANES * q, LANES)] for q in range(CL)]

        def norm_rows(n_rows, off):
            def node(nn, nvv):
                rec = plsc.load_gather(va, [nvv])
                for q in range(CL):
                    sl = pl.ds(LANES * q, LANES)
                    r0[nn, sl] = r0[nn, sl] * rec + bvecs[q]
                return nvv + 1

            lax.fori_loop(0, n_rows, node, jnp.full((LANES,), off, i32))

        for k in range(5):
            off = base + 128 * k

            @pl.when(off + 128 <= N)
            def _():
                pltpu.sync_copy(acc.at[pl.ds(off, 128)], r0)
                norm_rows(BLK, off)
                pltpu.sync_copy(r0, out_hbm.at[c, pl.ds(off, 128)])

        @pl.when(t == NS - 1)
        def _():
            pltpu.sync_copy(acc.at[pl.ds(N - 16, 16)], r0.at[pl.ds(0, 16)])
            norm_rows(16, N - 16)
            pltpu.sync_copy(r0.at[pl.ds(0, 16)],
                            out_hbm.at[c, pl.ds(N - 16, 16)])

    return sc_layer


def _tc_dense_fn(relu_in, Cout):
    def body(x_ref, w_ref, asr_ref, adr_ref, h_ref, a_ref, m_ref):
        xv = x_ref[...]
        if relu_in:
            xv = jnp.maximum(xv, 0.0)
        h = jnp.dot(xv, w_ref[...], preferred_element_type=jnp.float32)
        h_ref[...] = h
        a_s = jnp.sum(h * asr_ref[...][None, :], axis=1)
        a_d = jnp.sum(h * adr_ref[...][None, :], axis=1)
        a_ref[...] = jnp.concatenate([a_s[None, :], a_d[None, :]], axis=0)
        m = jnp.max(a_s) + jnp.max(a_d)
        m = jnp.where(m >= 0, m, 0.2 * m)
        m_ref[...] = jnp.full((LANES,), m, jnp.float32)

    return pl.pallas_call(
        body,
        out_shape=(jax.ShapeDtypeStruct((N, Cout), jnp.float32),
                   jax.ShapeDtypeStruct((2, N), jnp.float32),
                   jax.ShapeDtypeStruct((LANES,), jnp.float32)),
    )


_tc1 = _tc_dense_fn(False, 256)
_tc2 = _tc_dense_fn(True, 128)
_sc1a = _sc_layer_fn(4, 2, 0)
_sc1b = _sc_layer_fn(4, 2, 1)
_sc2 = _sc_layer_fn(2, 1, 0)


def kernel(x, edge_index, W1, att_src1, att_dst1, b1,
           W2, att_src2, att_dst2, b2):
    x = x.astype(jnp.float32)
    loop = jnp.arange(N, dtype=jnp.int32)
    pad = jnp.zeros((EPAD - ET,), jnp.int32)
    src3 = jnp.concatenate([edge_index[0], loop, pad]).reshape(NS, KB, BLK)
    dst3 = jnp.concatenate([edge_index[1], loop, pad]).reshape(NS, KB, BLK)
    b1q = b1.reshape(4, C)
    b1a = jnp.stack([b1q[0], b1q[2]])
    b1b = jnp.stack([b1q[1], b1q[3]])
    b2h = b2.reshape(2, C)

    h1, a1, m1 = _tc1(x, W1, att_src1, att_dst1)
    h1v = h1.reshape(4 * N, C)
    ya = _sc1a(h1v, a1, m1, src3, dst3, b1a)
    yb = _sc1b(h1v, a1, m1, src3, dst3, b1b)
    y1c = jnp.concatenate([ya[0], yb[0], ya[1], yb[1]], axis=1)
    h2, a2, m2 = _tc2(y1c, W2, att_src2, att_dst2)
    y2 = _sc2(h2.reshape(2 * N, C), a2, m2, src3, dst3, b2h)
    return jnp.concatenate([y2[0], y2[1]], axis=1)


# speedup vs baseline: 1.3855x; 1.3855x over previous
import functools

import jax
import jax.numpy as jnp
from jax import lax
from jax.experimental import pallas as pl
from jax.experimental.pallas import tpu as pltpu
from jax.experimental.pallas import tpu_sc as plsc

N = 10000
E = 320000
ET = E + N
NC = 2
NS = 16
LANES = 16
BLK = 128
NCH = 3
NR = 5
NQ = 11
KBC = NR * NQ
KB = NCH * KBC
EPT = KB * BLK
EPAD = NS * EPT
NPT = 640
C = 64
CL = C // LANES


def _sc_layer_fn(S, T, K):
    mesh = plsc.VectorSubcoreMesh(
        core_axis_name="c", subcore_axis_name="s",
        num_cores=NC, num_subcores=NS)

    @functools.partial(
        pl.kernel,
        out_type=jax.ShapeDtypeStruct((NC, N, C), jnp.float32),
        mesh=mesh,
        compiler_params=pltpu.CompilerParams(
            needs_layout_passes=False, use_tc_tiling_on_sc=False),
        scratch_types=[
            pltpu.VMEM((KBC, BLK), jnp.int32),
            pltpu.VMEM((KBC, BLK), jnp.int32),
            pltpu.VMEM((KBC, BLK), jnp.float32),
            pltpu.VMEM((N,), jnp.float32),
            pltpu.VMEM((N,), jnp.float32),
            pltpu.VMEM((BLK, C), jnp.float32),
            pltpu.VMEM((BLK, C), jnp.float32),
            pltpu.VMEM((BLK, C), jnp.float32),
            pltpu.VMEM((BLK, C), jnp.float32),
            pltpu.VMEM((BLK, C), jnp.float32),
            pltpu.VMEM((LANES,), jnp.float32),
            pltpu.VMEM_SHARED((N,), jnp.float32),
            pltpu.VMEM_SHARED((N, C), jnp.float32),
            pltpu.SemaphoreType.DMA,
            pltpu.SemaphoreType.DMA,
            pltpu.SemaphoreType.DMA,
            pltpu.SemaphoreType.DMA,
            pltpu.SemaphoreType.DMA,
            pltpu.SemaphoreType.DMA,
            pltpu.SemaphoreType.DMA,
            pltpu.SemaphoreType.DMA,
            pltpu.SemaphoreType.DMA,
            pltpu.SemaphoreType.DMA,
            pltpu.SemaphoreType.DMA,
        ],
    )
    def sc_layer(h_hbm, ab_hbm, m_hbm, src_hbm, dst_hbm, bias_hbm, out_hbm,
                 srcg, dstv, w, va, vb, r0, r1, r2, r3, r4, vmb, denom, acc,
                 g0, g1, g2, g3, g4, s0, s1, s2, s3, s4, dsem):
        i32 = jnp.int32
        f32 = jnp.float32
        c = lax.axis_index("c")
        t = lax.axis_index("s")
        giota = lax.iota(i32, LANES)
        base = t * NPT
        zero16 = jnp.zeros((LANES,), f32)
        bufs = [r0, r1, r2, r3, r4]
        gsems = [g0, g1, g2, g3, g4]
        ssems = [s0, s1, s2, s3, s4]

        for q in range(BLK // LANES):
            w[0, pl.ds(LANES * q, LANES)] = zero16

        def zrow(r, carry):
            for q in range(CL):
                r0[r, pl.ds(LANES * q, LANES)] = zero16
            return carry

        lax.fori_loop(0, BLK, zrow, 0)
        for k in range(5):
            off = base + 128 * k

            @pl.when(off + 128 <= N)
            def _():
                pltpu.sync_copy(w.at[0], denom.at[pl.ds(off, 128)])
                pltpu.sync_copy(r0, acc.at[pl.ds(off, 128)])

        @pl.when(t == NS - 1)
        def _():
            pltpu.sync_copy(w.at[0, pl.ds(0, 16)],
                            denom.at[pl.ds(N - 16, 16)])
            pltpu.sync_copy(r0.at[pl.ds(0, 16)],
                            acc.at[pl.ds(N - 16, 16)])

        pltpu.sync_copy(ab_hbm.at[0], va)
        pltpu.sync_copy(ab_hbm.at[1], vb)
        pltpu.sync_copy(m_hbm, vmb)
        mb = vmb[...]

        plsc.subcore_barrier()

        def g_issue(j, buf, sem):
            pltpu.async_copy(h_hbm.at[srcg.at[j]], buf, sem)

        def g_drain(buf, sem):
            pltpu.make_async_copy(h_hbm.at[srcg.at[0]], buf, sem).wait()

        def s_issue(j, buf, sem):
            pltpu.async_copy(buf, acc.at[dstv.at[j]], sem, add=True)

        def s_drain(buf, sem):
            pltpu.make_async_copy(buf, acc.at[pl.ds(0, BLK)], sem).wait()

        def d_issue(j):
            pltpu.async_copy(w.at[j], denom.at[dstv.at[j]], dsem, add=True)

        def d_drain():
            pltpu.make_async_copy(w.at[0], denom.at[pl.ds(0, BLK)],
                                  dsem).wait()

        def scale(buf, j):
            jv = jnp.full((LANES,), j, i32)

            def estep(i, ev):
                e0 = i * 4
                for u in range(4):
                    av = plsc.load_gather(w, [jv, ev + u])
                    for q in range(CL):
                        sl = pl.ds(LANES * q, LANES)
                        buf[e0 + u, sl] = buf[e0 + u, sl] * av
                return ev + 4

            lax.fori_loop(0, BLK // 4, estep, jnp.zeros((LANES,), i32))

        for ch in range(NCH):
            cb = ch * KBC
            pltpu.sync_copy(src_hbm.at[t, pl.ds(cb, KBC)], srcg)
            pltpu.sync_copy(dst_hbm.at[t, pl.ds(cb, KBC)], dstv)
            ebase = t * EPT + cb * BLK

            def escomp(j, carry):
                jb = ebase + j * BLK
                for k in range(BLK // LANES):
                    sl = pl.ds(LANES * k, LANES)
                    sv = srcg[j, sl]
                    dv = dstv[j, sl]
                    e = (plsc.load_gather(va, [sv])
                         + plsc.load_gather(vb, [dv]))
                    e = jnp.where(e >= 0, e, 0.2 * e)
                    ex = jnp.exp(e - mb)
                    gid = jb + LANES * k + giota
                    ex = jnp.where(gid < ET, ex, 0.0)
                    w[j, sl] = ex
                    srcg[j, sl] = sv * S + c * T + K
                return carry

            lax.fori_loop(0, KBC, escomp, 0)

            g_issue(0, r0, g0)
            g_issue(1, r1, g1)
            g_issue(2, r2, g2)

            def ring(qq, carry):
                for i in range(NR):
                    j = NR * qq + i
                    buf, gs, ss = bufs[i], gsems[i], ssems[i]
                    nb = (i + 3) % NR
                    g_drain(buf, gs)
                    scale(buf, j)
                    s_issue(j, buf, ss)
                    d_issue(j)

                    @pl.when(j >= 2)
                    def _():
                        s_drain(bufs[nb], ssems[nb])
                        d_drain()

                    @pl.when(j + 3 < KBC)
                    def _():
                        g_issue(j + 3, bufs[nb], gsems[nb])
                return carry

            lax.fori_loop(0, NQ, ring, 0)
            s_drain(bufs[(KBC - 2) % NR], ssems[(KBC - 2) % NR])
            s_drain(bufs[(KBC - 1) % NR], ssems[(KBC - 1) % NR])
            d_drain()
            d_drain()

        plsc.subcore_barrier()

        pltpu.sync_copy(denom, va)
        nv = jnp.where(t == NS - 1, 25, 40)

        def rstep(r, carry):
            sl = pl.ds(base + LANES * r, LANES)
            va[sl] = 1.0 / (va[sl] + 1e-16)
            return carry

        lax.fori_loop(0, nv, rstep, 0)
        pltpu.sync_copy(bias_hbm.at[c], r1.at[0])
        bvecs = [r1[0, pl.ds(LANES * q, LANES)] for q in range(CL)]

        def norm_rows(n_rows, off):
            def node(nn, nvv):
                rec = plsc.load_gather(va, [nvv])
                for q in range(CL):
                    sl = pl.ds(LANES * q, LANES)
                    r0[nn, sl] = r0[nn, sl] * rec + bvecs[q]
                return nvv + 1

            lax.fori_loop(0, n_rows, node, jnp.full((LANES,), off, i32))

        for k in range(5):
            off = base + 128 * k

            @pl.when(off + 128 <= N)
            def _():
                pltpu.sync_copy(acc.at[pl.ds(off, 128)], r0)
                norm_rows(BLK, off)
                pltpu.sync_copy(r0, out_hbm.at[c, pl.ds(off, 128)])

        @pl.when(t == NS - 1)
        def _():
            pltpu.sync_copy(acc.at[pl.ds(N - 16, 16)], r0.at[pl.ds(0, 16)])
            norm_rows(16, N - 16)
            pltpu.sync_copy(r0.at[pl.ds(0, 16)],
                            out_hbm.at[c, pl.ds(N - 16, 16)])

    return sc_layer


def _tc_dense_fn(relu_in, Cout):
    def body(x_ref, w_ref, asr_ref, adr_ref, h_ref, a_ref, m_ref):
        xv = x_ref[...]
        if relu_in:
            xv = jnp.maximum(xv, 0.0)
        h = jnp.dot(xv, w_ref[...], preferred_element_type=jnp.float32)
        h_ref[...] = h
        a_s = jnp.sum(h * asr_ref[...][None, :], axis=1)
        a_d = jnp.sum(h * adr_ref[...][None, :], axis=1)
        a_ref[...] = jnp.concatenate([a_s[None, :], a_d[None, :]], axis=0)
        m = jnp.max(a_s) + jnp.max(a_d)
        m = jnp.where(m >= 0, m, 0.2 * m)
        m_ref[...] = jnp.full((LANES,), m, jnp.float32)

    return pl.pallas_call(
        body,
        out_shape=(jax.ShapeDtypeStruct((N, Cout), jnp.float32),
                   jax.ShapeDtypeStruct((2, N), jnp.float32),
                   jax.ShapeDtypeStruct((LANES,), jnp.float32)),
    )


_tc1 = _tc_dense_fn(False, 256)
_tc2 = _tc_dense_fn(True, 128)
_sc1a = _sc_layer_fn(4, 2, 0)
_sc1b = _sc_layer_fn(4, 2, 1)
_sc2 = _sc_layer_fn(2, 1, 0)


def kernel(x, edge_index, W1, att_src1, att_dst1, b1,
           W2, att_src2, att_dst2, b2):
    x = x.astype(jnp.float32)
    loop = jnp.arange(N, dtype=jnp.int32)
    pad = jnp.zeros((EPAD - ET,), jnp.int32)
    src3 = jnp.concatenate([edge_index[0], loop, pad]).reshape(NS, KB, BLK)
    dst3 = jnp.concatenate([edge_index[1], loop, pad]).reshape(NS, KB, BLK)
    b1q = b1.reshape(4, C)
    b1a = jnp.stack([b1q[0], b1q[2]])
    b1b = jnp.stack([b1q[1], b1q[3]])
    b2h = b2.reshape(2, C)

    h1, a1, m1 = _tc1(x, W1, att_src1, att_dst1)
    h1v = h1.reshape(4 * N, C)
    ya = _sc1a(h1v, a1, m1, src3, dst3, b1a)
    yb = _sc1b(h1v, a1, m1, src3, dst3, b1b)
    y1c = jnp.concatenate([ya[0], yb[0], ya[1], yb[1]], axis=1)
    h2, a2, m2 = _tc2(y1c, W2, att_src2, att_dst2)
    y2 = _sc2(h2.reshape(2 * N, C), a2, m2, src3, dst3, b2h)
    return jnp.concatenate([y2[0], y2[1]], axis=1)


# speedup vs baseline: 1.8844x; 1.3601x over previous
import functools

import jax
import jax.numpy as jnp
from jax import lax
from jax.experimental import pallas as pl
from jax.experimental.pallas import tpu as pltpu
from jax.experimental.pallas import tpu_sc as plsc

N = 10000
E = 320000
ET = E + N
NC = 2
NS = 16
LANES = 16
BLK = 128
NCH = 3
KBC = 54
KB = NCH * KBC
NP = KBC // 2
EPT = KB * BLK
EPAD = NS * EPT
NPT = 640
C = 64
CL = C // LANES


def _sc_layer_fn(S, T, K):
    mesh = plsc.VectorSubcoreMesh(
        core_axis_name="c", subcore_axis_name="s",
        num_cores=NC, num_subcores=NS)

    @functools.partial(
        pl.kernel,
        out_type=jax.ShapeDtypeStruct((NC, N, C), jnp.float32),
        mesh=mesh,
        compiler_params=pltpu.CompilerParams(
            needs_layout_passes=False, use_tc_tiling_on_sc=False),
        scratch_types=[
            pltpu.VMEM((KBC, BLK), jnp.int32),
            pltpu.VMEM((KBC, BLK), jnp.int32),
            pltpu.VMEM((KBC, BLK), jnp.float32),
            pltpu.VMEM((N,), jnp.float32),
            pltpu.VMEM((N,), jnp.float32),
            pltpu.VMEM((BLK, C), jnp.float32),
            pltpu.VMEM((BLK, C), jnp.float32),
            pltpu.VMEM((LANES,), jnp.float32),
            pltpu.VMEM_SHARED((N,), jnp.float32),
            pltpu.VMEM_SHARED((N, C), jnp.float32),
            pltpu.SemaphoreType.DMA,
            pltpu.SemaphoreType.DMA,
            pltpu.SemaphoreType.DMA,
            pltpu.SemaphoreType.DMA,
            pltpu.SemaphoreType.DMA,
        ],
    )
    def sc_layer(h_hbm, ab_hbm, m_hbm, src_hbm, dst_hbm, bias_hbm, out_hbm,
                 srcg, dstv, w, va, vb, rowsA, rowsB, vmb, denom, acc,
                 gsA, gsB, ssA, ssB, dsem):
        i32 = jnp.int32
        f32 = jnp.float32
        c = lax.axis_index("c")
        t = lax.axis_index("s")
        giota = lax.iota(i32, LANES)
        base = t * NPT
        zero16 = jnp.zeros((LANES,), f32)

        for q in range(BLK // LANES):
            w[0, pl.ds(LANES * q, LANES)] = zero16

        def zrow(r, carry):
            for q in range(CL):
                rowsA[r, pl.ds(LANES * q, LANES)] = zero16
            return carry

        lax.fori_loop(0, BLK, zrow, 0)
        for k in range(5):
            off = base + 128 * k

            @pl.when(off + 128 <= N)
            def _():
                pltpu.sync_copy(w.at[0], denom.at[pl.ds(off, 128)])
                pltpu.sync_copy(rowsA, acc.at[pl.ds(off, 128)])

        @pl.when(t == NS - 1)
        def _():
            pltpu.sync_copy(w.at[0, pl.ds(0, 16)],
                            denom.at[pl.ds(N - 16, 16)])
            pltpu.sync_copy(rowsA.at[pl.ds(0, 16)],
                            acc.at[pl.ds(N - 16, 16)])

        pltpu.sync_copy(ab_hbm.at[0], va)
        pltpu.sync_copy(ab_hbm.at[1], vb)
        pltpu.sync_copy(m_hbm, vmb)
        mb = vmb[...]

        plsc.subcore_barrier()

        def g_issue(j, buf, sem):
            pltpu.async_copy(h_hbm.at[srcg.at[j]], buf, sem)

        def g_drain(buf, sem):
            pltpu.make_async_copy(h_hbm.at[srcg.at[0]], buf, sem).wait()

        def s_issue(j, buf, sem):
            pltpu.async_copy(buf, acc.at[dstv.at[j]], sem, add=True)

        def s_drain(buf, sem):
            pltpu.make_async_copy(buf, acc.at[pl.ds(0, BLK)], sem).wait()

        def d_issue(j):
            pltpu.async_copy(w.at[j], denom.at[dstv.at[j]], dsem, add=True)

        def d_drain():
            pltpu.make_async_copy(w.at[0], denom.at[pl.ds(0, BLK)],
                                  dsem).wait()

        def scale(buf, j):
            jv = jnp.full((LANES,), j, i32)

            def estep(i, ev):
                e0 = i * 4
                for u in range(4):
                    av = plsc.load_gather(w, [jv, ev + u])
                    for q in range(CL):
                        sl = pl.ds(LANES * q, LANES)
                        buf[e0 + u, sl] = buf[e0 + u, sl] * av
                return ev + 4

            lax.fori_loop(0, BLK // 4, estep, jnp.zeros((LANES,), i32))

        for ch in range(NCH):
            cb = ch * KBC
            pltpu.sync_copy(src_hbm.at[t, pl.ds(cb, KBC)], srcg)
            pltpu.sync_copy(dst_hbm.at[t, pl.ds(cb, KBC)], dstv)
            ebase = t * EPT + cb * BLK

            def escomp(j, carry):
                jb = ebase + j * BLK
                for k in range(BLK // LANES):
                    sl = pl.ds(LANES * k, LANES)
                    sv = srcg[j, sl]
                    dv = dstv[j, sl]
                    e = (plsc.load_gather(va, [sv])
                         + plsc.load_gather(vb, [dv]))
                    e = jnp.where(e >= 0, e, 0.2 * e)
                    ex = jnp.exp(e - mb)
                    gid = jb + LANES * k + giota
                    ex = jnp.where(gid < ET, ex, 0.0)
                    w[j, sl] = ex
                    srcg[j, sl] = sv * S + c * T + K
                return carry

            lax.fori_loop(0, KBC, escomp, 0)

            g_issue(0, rowsA, gsA)

            def pair(p, carry):
                j0 = 2 * p
                j1 = 2 * p + 1

                @pl.when(p > 0)
                def _():
                    s_drain(rowsB, ssB)
                    d_drain()
                    d_drain()

                g_issue(j1, rowsB, gsB)
                g_drain(rowsA, gsA)
                scale(rowsA, j0)
                s_issue(j0, rowsA, ssA)
                d_issue(j0)
                g_drain(rowsB, gsB)
                s_drain(rowsA, ssA)

                @pl.when(p < NP - 1)
                def _():
                    g_issue(j0 + 2, rowsA, gsA)

                scale(rowsB, j1)
                s_issue(j1, rowsB, ssB)
                d_issue(j1)
                return carry

            lax.fori_loop(0, NP, pair, 0)
            s_drain(rowsB, ssB)
            d_drain()
            d_drain()

        plsc.subcore_barrier()

        pltpu.sync_copy(denom, va)
        nv = jnp.where(t == NS - 1, 25, 40)

        def rstep(r, carry):
            sl = pl.ds(base + LANES * r, LANES)
            va[sl] = 1.0 / (va[sl] + 1e-16)
            return carry

        lax.fori_loop(0, nv, rstep, 0)
        pltpu.sync_copy(bias_hbm.at[c], rowsB.at[0])
        bvecs = [rowsB[0, pl.ds(LANES * q, LANES)] for q in range(CL)]

        def norm_rows(n_rows, off):
            def node(nn, nvv):
                rec = plsc.load_gather(va, [nvv])
                for q in range(CL):
                    sl = pl.ds(LANES * q, LANES)
                    rowsA[nn, sl] = rowsA[nn, sl] * rec + bvecs[q]
                return nvv + 1

            lax.fori_loop(0, n_rows, node, jnp.full((LANES,), off, i32))

        for k in range(5):
            off = base + 128 * k

            @pl.when(off + 128 <= N)
            def _():
                pltpu.sync_copy(acc.at[pl.ds(off, 128)], rowsA)
                norm_rows(BLK, off)
                pltpu.sync_copy(rowsA, out_hbm.at[c, pl.ds(off, 128)])

        @pl.when(t == NS - 1)
        def _():
            pltpu.sync_copy(acc.at[pl.ds(N - 16, 16)], rowsA.at[pl.ds(0, 16)])
            norm_rows(16, N - 16)
            pltpu.sync_copy(rowsA.at[pl.ds(0, 16)],
                            out_hbm.at[c, pl.ds(N - 16, 16)])

    return sc_layer


def _tc_dense_fn(relu_in, Cout):
    def body(x_ref, w_ref, asr_ref, adr_ref, h_ref, a_ref, m_ref):
        xv = x_ref[...]
        if relu_in:
            xv = jnp.maximum(xv, 0.0)
        h = jnp.dot(xv, w_ref[...], preferred_element_type=jnp.float32)
        h_ref[...] = h
        a_s = jnp.sum(h * asr_ref[...][None, :], axis=1)
        a_d = jnp.sum(h * adr_ref[...][None, :], axis=1)
        a_ref[...] = jnp.concatenate([a_s[None, :], a_d[None, :]], axis=0)
        m = jnp.max(a_s) + jnp.max(a_d)
        m = jnp.where(m >= 0, m, 0.2 * m)
        m_ref[...] = jnp.full((LANES,), m, jnp.float32)

    return pl.pallas_call(
        body,
        out_shape=(jax.ShapeDtypeStruct((N, Cout), jnp.float32),
                   jax.ShapeDtypeStruct((2, N), jnp.float32),
                   jax.ShapeDtypeStruct((LANES,), jnp.float32)),
    )


_tc1 = _tc_dense_fn(False, 256)
_tc2 = _tc_dense_fn(True, 128)
_sc1a = _sc_layer_fn(4, 2, 0)
_sc1b = _sc_layer_fn(4, 2, 1)
_sc2 = _sc_layer_fn(2, 1, 0)


def kernel(x, edge_index, W1, att_src1, att_dst1, b1,
           W2, att_src2, att_dst2, b2):
    x = x.astype(jnp.float32)
    loop = jnp.arange(N, dtype=jnp.int32)
    pad = jnp.zeros((EPAD - ET,), jnp.int32)
    src3 = jnp.concatenate([edge_index[0], loop, pad]).reshape(NS, KB, BLK)
    dst3 = jnp.concatenate([edge_index[1], loop, pad]).reshape(NS, KB, BLK)
    b1q = b1.reshape(4, C)
    b1a = jnp.stack([b1q[0], b1q[2]])
    b1b = jnp.stack([b1q[1], b1q[3]])
    b2h = b2.reshape(2, C)

    h1, a1, m1 = _tc1(x, W1, att_src1, att_dst1)
    h1v = h1.reshape(4 * N, C)
    ya = _sc1a(h1v, a1, m1, src3, dst3, b1a)
    yb = _sc1b(h1v, a1, m1, src3, dst3, b1b)
    y1c = jnp.concatenate([ya[0], yb[0], ya[1], yb[1]], axis=1)
    h2, a2, m2 = _tc2(y1c, W2, att_src2, att_dst2)
    y2 = _sc2(h2.reshape(2 * N, C), a2, m2, src3, dst3, b2h)
    return jnp.concatenate([y2[0], y2[1]], axis=1)
